# baseline (speedup 1.0000x reference)
import functools

import jax
import jax.numpy as jnp
from jax import lax
from jax.experimental import pallas as pl
from jax.experimental.pallas import tpu as pltpu
from jax.experimental.pallas import tpu_sc as plsc

N_NODES = 10000
N_EDGES = 320000
FEAT = 128
N_GRAPHS = 64

NC, NS = 2, 16
NW = NC * NS
EPW = N_EDGES // NW
CH = 40
NCHUNK = EPW // CH
N_PAD = 10240
RPT = N_PAD // NS
ZR = 128
VEC = 16


def _sc_aggr_body(h_hbm, e_hbm, srcr_hbm, dstr_hbm, out_hbm,
                  aggr_sh, c0, c1, d0, d1,
                  x0, x1, e0, e1, m0, m1,
                  g0, g1, l0, l1, s0, s1, p0, p1, q0, q1):
    cid = lax.axis_index("c")
    sid = lax.axis_index("s")
    wid = cid * NS + sid
    ebase = wid * EPW

    xbufs, ebufs, mbufs = (x0, x1), (e0, e1), (m0, m1)
    cbufs, dbufs = (c0, c1), (d0, d1)
    gsems, lsems, ssems = (g0, g1), (l0, l1), (s0, s1)
    psems, qsems = (p0, p1), (q0, q1)

    z = jnp.zeros((VEC,), jnp.float32)

    def zrow(r, carry):
        for c in range(FEAT // VEC):
            m0[r, pl.ds(c * VEC, VEC)] = z
        return carry
    lax.fori_loop(0, CH, zrow, 0)

    for k in range(RPT // CH):
        pltpu.sync_copy(m0, aggr_sh.at[pl.ds(sid * RPT + k * CH, CH)])
    plsc.subcore_barrier()

    def issue_src(T, b):
        pltpu.async_copy(srcr_hbm.at[wid, T], cbufs[b], qsems[b])

    def issue(b):
        pltpu.make_async_copy(srcr_hbm.at[wid, 0], cbufs[b], qsems[b]).wait()
        pltpu.async_copy(h_hbm.at[cbufs[b]], xbufs[b], gsems[b])

    def issue_e(T, b):
        pltpu.async_copy(e_hbm.at[pl.ds(ebase + T * CH, CH)], ebufs[b],
                         lsems[b])

    def drain_scatter(b):
        pltpu.make_async_copy(e_hbm.at[pl.ds(0, CH)], mbufs[b],
                              ssems[b]).wait()

    def process(T, b):
        pltpu.make_async_copy(e_hbm.at[pl.ds(0, CH)], xbufs[b],
                              gsems[b]).wait()
        pltpu.make_async_copy(e_hbm.at[pl.ds(0, CH)], ebufs[b],
                              lsems[b]).wait()

        @pl.when(T >= 2)
        def _():
            drain_scatter(b)

        pltpu.async_copy(dstr_hbm.at[wid, T], dbufs[b], psems[b])

        @pl.when(T + 2 < NCHUNK)
        def _():
            issue_src(T + 2, b)

        def row(r, carry):
            for c in range(FEAT // VEC):
                sl = pl.ds(c * VEC, VEC)
                mbufs[b][r, sl] = jnp.maximum(
                    xbufs[b][r, sl] + ebufs[b][r, sl], 0.0)
            return carry
        lax.fori_loop(0, CH, row, 0, unroll=4)

        @pl.when(T + 2 < NCHUNK)
        def _():
            issue(b)
            issue_e(T + 2, b)

        pltpu.make_async_copy(dstr_hbm.at[wid, T], dbufs[b], psems[b]).wait()
        pltpu.async_copy(mbufs[b], aggr_sh.at[dbufs[b]], ssems[b],
                         add=True)

    issue_src(0, 0)
    issue(0)
    issue_e(0, 0)
    issue_src(1, 1)
    issue(1)
    issue_e(1, 1)

    def pair(i, carry):
        t = i * 2
        process(t, 0)
        process(t + 1, 1)
        return carry
    lax.fori_loop(0, NCHUNK // 2, pair, 0)

    drain_scatter(0)
    drain_scatter(1)

    plsc.subcore_barrier()
    pltpu.sync_copy(aggr_sh.at[pl.ds(sid * RPT, RPT)],
                    out_hbm.at[cid, pl.ds(sid * RPT, RPT)])


@functools.cache
def _make_sc_aggr():
    return pl.kernel(
        _sc_aggr_body,
        out_type=jax.ShapeDtypeStruct((NC, N_PAD, FEAT), jnp.float32),
        mesh=plsc.VectorSubcoreMesh(core_axis_name="c", subcore_axis_name="s",
                                    num_cores=NC, num_subcores=NS),
        scratch_types=[
            pltpu.VMEM_SHARED((N_PAD, FEAT), jnp.float32),
            pltpu.VMEM((CH,), jnp.int32),
            pltpu.VMEM((CH,), jnp.int32),
            pltpu.VMEM((CH,), jnp.int32),
            pltpu.VMEM((CH,), jnp.int32),
            pltpu.VMEM((CH, FEAT), jnp.float32),
            pltpu.VMEM((CH, FEAT), jnp.float32),
            pltpu.VMEM((CH, FEAT), jnp.float32),
            pltpu.VMEM((CH, FEAT), jnp.float32),
            pltpu.VMEM((CH, FEAT), jnp.float32),
            pltpu.VMEM((CH, FEAT), jnp.float32),
            pltpu.SemaphoreType.DMA,
            pltpu.SemaphoreType.DMA,
            pltpu.SemaphoreType.DMA,
            pltpu.SemaphoreType.DMA,
            pltpu.SemaphoreType.DMA,
            pltpu.SemaphoreType.DMA,
            pltpu.SemaphoreType.DMA,
            pltpu.SemaphoreType.DMA,
            pltpu.SemaphoreType.DMA,
            pltpu.SemaphoreType.DMA,
        ],
    )


def _sc_aggr(h, e, src, dst):
    srcr = src.reshape(NW, NCHUNK, CH)
    dstr = dst.reshape(NW, NCHUNK, CH)
    return _make_sc_aggr()(h, e, srcr, dstr)[:, :N_NODES, :]


def _edge_mlp_body(ea_ref, w_ref, b_ref, out_ref):
    out_ref[...] = jnp.dot(ea_ref[...], w_ref[...],
                           preferred_element_type=jnp.float32) + b_ref[...]


_EB = 2000


def _edge_mlp(ea, W, b):
    ed = ea.shape[1]
    return pl.pallas_call(
        _edge_mlp_body,
        grid=(N_EDGES // _EB,),
        in_specs=[
            pl.BlockSpec((_EB, ed), lambda i: (i, 0)),
            pl.BlockSpec((ed, FEAT), lambda i: (0, 0)),
            pl.BlockSpec((1, FEAT), lambda i: (0, 0)),
        ],
        out_specs=pl.BlockSpec((_EB, FEAT), lambda i: (i, 0)),
        out_shape=jax.ShapeDtypeStruct((N_EDGES, FEAT), jnp.float32),
    )(ea, W, b.reshape(1, FEAT))


def _node_body(h_ref, a_ref, w_ref, b_ref, out_ref):
    s = h_ref[...] + a_ref[0] + a_ref[1]
    out_ref[...] = jax.nn.gelu(
        jnp.dot(s, w_ref[...], preferred_element_type=jnp.float32)
        + b_ref[...])


_NB = 2000


def _node_update(h, parts, W, b):
    return pl.pallas_call(
        _node_body,
        grid=(N_NODES // _NB,),
        in_specs=[
            pl.BlockSpec((_NB, FEAT), lambda i: (i, 0)),
            pl.BlockSpec((NC, _NB, FEAT), lambda i: (0, i, 0)),
            pl.BlockSpec((FEAT, FEAT), lambda i: (0, 0)),
            pl.BlockSpec((1, FEAT), lambda i: (0, 0)),
        ],
        out_specs=pl.BlockSpec((_NB, FEAT), lambda i: (i, 0)),
        out_shape=jax.ShapeDtypeStruct((N_NODES, FEAT), jnp.float32),
    )(h, parts, W, b.reshape(1, FEAT))


def _head_body(h_ref, batch_ref, w1_ref, b1_ref, w2_ref, b2_ref, out_ref):
    onehot = (batch_ref[...] ==
              lax.broadcasted_iota(jnp.int32, (1, N_GRAPHS), 1)
              ).astype(jnp.float32)
    sums = lax.dot_general(onehot, h_ref[...], (((0,), (0,)), ((), ())),
                           preferred_element_type=jnp.float32)
    counts = jnp.sum(onehot, axis=0)
    pooled = sums / jnp.maximum(counts, 1.0)[:, None]
    t = jax.nn.gelu(jnp.dot(pooled, w1_ref[...],
                            preferred_element_type=jnp.float32) + b1_ref[...])
    out_ref[...] = jnp.dot(t, w2_ref[...],
                           preferred_element_type=jnp.float32) + b2_ref[...]


def _head(h, batch, fc1_W, fc1_b, fc2_W, fc2_b):
    return pl.pallas_call(
        _head_body,
        out_shape=jax.ShapeDtypeStruct((N_GRAPHS, 1), jnp.float32),
    )(h, batch.reshape(N_NODES, 1), fc1_W, fc1_b.reshape(1, 64),
      fc2_W, fc2_b.reshape(1, 1))


def kernel(x, edge_index, batch, edge_attr,
           We0, be0, Wn0, bn0, We1, be1, Wn1, bn1, We2, be2, Wn2, bn2,
           fc1_W, fc1_b, fc2_W, fc2_b):
    src = edge_index[0]
    dst = edge_index[1]
    h = x
    for We, be, Wn, bn in ((We0, be0, Wn0, bn0),
                           (We1, be1, Wn1, bn1),
                           (We2, be2, Wn2, bn2)):
        e = _edge_mlp(edge_attr, We, be)
        parts = _sc_aggr(h, e, src, dst)
        h = _node_update(h, parts, Wn, bn)
    return _head(h, batch, fc1_W, fc1_b, fc2_W, fc2_b)
